# Initial kernel scaffold; baseline (speedup 1.0000x reference)
import jax
import jax.numpy as jnp
from jax import lax
from jax.experimental import pallas as pl
from jax.experimental.pallas import tpu as pltpu
from jax.experimental.pallas import tpu_sc as plsc

_NC = 2
_NS = 16
_NW = _NC * _NS
_CHUNK = 80
_F32 = jnp.float32


def _degcnt_body(src_hbm, dst_hbm, ew_hbm, zeros_hbm, out_hbm,
                 src_sl, dst_sl, ew_sl, ones_v, deg_s, cnt_s):
    n = out_hbm.shape[2]
    cpt = src_hbm.shape[1]
    c = lax.axis_index("c")
    s = lax.axis_index("s")
    w = s * _NC + c

    @pl.when(s == 0)
    def _zero():
        for t in range(n // 2000):
            pltpu.sync_copy(zeros_hbm, deg_s.at[pl.ds(2000 * t, 2000)])
            pltpu.sync_copy(zeros_hbm, cnt_s.at[pl.ds(2000 * t, 2000)])

    one16 = jnp.ones((16,), _F32)
    for t in range(_CHUNK // 16):
        ones_v[pl.ds(16 * t, 16)] = one16
    plsc.subcore_barrier()

    pltpu.sync_copy(src_hbm.at[w], src_sl)
    pltpu.sync_copy(dst_hbm.at[w], dst_sl)
    pltpu.sync_copy(ew_hbm.at[w], ew_sl)

    def chunk(j, _):
        pltpu.sync_copy(ew_sl.at[j], deg_s.at[src_sl.at[j]], add=True)
        pltpu.sync_copy(ones_v, cnt_s.at[dst_sl.at[j]], add=True)
        return ()

    lax.fori_loop(0, cpt, chunk, ())
    plsc.subcore_barrier()

    @pl.when(s == 0)
    def _writeback():
        pltpu.sync_copy(deg_s, out_hbm.at[c, 0])
        pltpu.sync_copy(cnt_s, out_hbm.at[c, 1])


def _edge_body(src_hbm, dst_hbm, ew_hbm, y_hbm, zeros_hbm, out_hbm,
               src_sl, dst_sl, ew_sl, rows_v, acc_s, sem):
    n = y_hbm.shape[0]
    cpt = src_hbm.shape[1]
    rpt = n // _NS
    c = lax.axis_index("c")
    s = lax.axis_index("s")
    w = s * _NC + c

    zrows = zeros_hbm.shape[0]
    for t in range(rpt // zrows):
        pltpu.sync_copy(zeros_hbm, acc_s.at[pl.ds(rpt * s + zrows * t, zrows)])
    plsc.subcore_barrier()

    pltpu.sync_copy(src_hbm.at[w], src_sl)
    pltpu.sync_copy(dst_hbm.at[w], dst_sl)
    pltpu.sync_copy(ew_hbm.at[w], ew_sl)

    def chunk(j, _):
        pltpu.async_copy(y_hbm.at[src_sl.at[j]], rows_v, sem).wait()
        for g in range(_CHUNK // 16):
            ew16 = ew_sl[j, pl.ds(g * 16, 16)]
            for kk in range(16):
                bc = ew16.at[jnp.full((16,), kk, jnp.int32)].get(
                    mode="promise_in_bounds")
                i = g * 16 + kk
                for q in range(4):
                    sl = pl.ds(q * 16, 16)
                    rows_v[i, sl] = rows_v[i, sl] * bc
        pltpu.sync_copy(rows_v, acc_s.at[dst_sl.at[j]], add=True)
        return ()

    lax.fori_loop(0, cpt, chunk, ())
    plsc.subcore_barrier()
    pltpu.sync_copy(acc_s.at[pl.ds(rpt * s, rpt)],
                    out_hbm.at[c, pl.ds(rpt * s, rpt)])


def _sc_degcnt(src3, dst3, ew3, n):
    cpt, chunk = src3.shape[1], src3.shape[2]
    mesh = plsc.VectorSubcoreMesh(core_axis_name="c", subcore_axis_name="s")
    fn = pl.kernel(
        _degcnt_body,
        out_type=jax.ShapeDtypeStruct((_NC, 2, n), _F32),
        mesh=mesh,
        scratch_types=[
            pltpu.VMEM((cpt, chunk), jnp.int32),
            pltpu.VMEM((cpt, chunk), jnp.int32),
            pltpu.VMEM((cpt, chunk), _F32),
            pltpu.VMEM((chunk,), _F32),
            pltpu.VMEM_SHARED((n,), _F32),
            pltpu.VMEM_SHARED((n,), _F32),
        ],
    )
    zeros = jnp.zeros((2000,), _F32)
    return fn(src3, dst3, ew3, zeros)


def _sc_edge_pass(src3, dst3, ew3, y):
    n = y.shape[0]
    cpt, chunk = src3.shape[1], src3.shape[2]
    mesh = plsc.VectorSubcoreMesh(core_axis_name="c", subcore_axis_name="s")
    fn = pl.kernel(
        _edge_body,
        out_type=jax.ShapeDtypeStruct((_NC, n, 128), _F32),
        mesh=mesh,
        scratch_types=[
            pltpu.VMEM((cpt, chunk), jnp.int32),
            pltpu.VMEM((cpt, chunk), jnp.int32),
            pltpu.VMEM((cpt, chunk), _F32),
            pltpu.VMEM((chunk, 128), _F32),
            pltpu.VMEM_SHARED((n, 128), _F32),
            pltpu.SemaphoreType.DMA,
        ],
    )
    zeros = jnp.zeros((125, 128), _F32)
    return fn(src3, dst3, ew3, y, zeros)


def _leaky(v):
    return jnp.where(v >= 0, v, 0.01 * v)


def _dis_invcnt(dc):
    deg = dc[:, 0:1] + dc[:, 2:3]
    cnt = dc[:, 1:2] + dc[:, 3:4]
    dis = jnp.where(deg > 0, lax.rsqrt(jnp.where(deg > 0, deg, 1.0)), 0.0)
    inv_cnt = 1.0 / jnp.clip(cnt, 1.0, None)
    return dis, inv_cnt


def _ypq(xx, dis, cW0, cW1, cb, sWl, sWr):
    u = jnp.dot(xx, cW1, preferred_element_type=_F32) * dis
    v = jnp.dot(xx, sWl, preferred_element_type=_F32)
    p = jnp.dot(xx, cW0, preferred_element_type=_F32) + cb
    q = jnp.dot(xx, sWr, preferred_element_type=_F32)
    return jnp.concatenate([u, v], axis=1), jnp.concatenate([p, q], axis=1)


def _combine(pq, S, dis, inv_cnt, sbl):
    sa = S[0, :, :64] + S[1, :, :64]
    sb = S[0, :, 64:] + S[1, :, 64:]
    o1 = _leaky(pq[:, :64] - dis * sa)
    o2 = _leaky(inv_cnt * sb + sbl + pq[:, 64:])
    return o1 + o2


def _tcA_body(x_ref, dc_ref, pW_ref, pb_ref, cW0_ref, cW1_ref, cb_ref,
              sWl_ref, sWr_ref, y_ref, pq_ref):
    dis, _ = _dis_invcnt(dc_ref[...])
    xx = jnp.dot(x_ref[...], pW_ref[...], preferred_element_type=_F32)
    xx = xx + pb_ref[...]
    y, pq = _ypq(xx, dis, cW0_ref[...], cW1_ref[...], cb_ref[...],
                 sWl_ref[...], sWr_ref[...])
    y_ref[...] = y
    pq_ref[...] = pq


def _tcB_body(pq_ref, s_ref, dc_ref, sbl_ref, lW_ref, lb_ref,
              pW_ref, pb_ref, cW0_ref, cW1_ref, cb_ref, sWl_ref, sWr_ref,
              y_ref, pq1_ref):
    dis, inv_cnt = _dis_invcnt(dc_ref[...])
    cur = _combine(pq_ref[...], s_ref[...], dis, inv_cnt, sbl_ref[...])
    cur = jnp.dot(cur, lW_ref[...], preferred_element_type=_F32) + lb_ref[...]
    xx = jnp.dot(cur, pW_ref[...], preferred_element_type=_F32) + pb_ref[...]
    y, pq = _ypq(xx, dis, cW0_ref[...], cW1_ref[...], cb_ref[...],
                 sWl_ref[...], sWr_ref[...])
    y_ref[...] = y
    pq1_ref[...] = pq


def _tcC_body(pq_ref, s_ref, dc_ref, sbl_ref, lW_ref, lb_ref,
              clsW_ref, clsb_ref, out_ref):
    dis, inv_cnt = _dis_invcnt(dc_ref[...])
    cur = _combine(pq_ref[...], s_ref[...], dis, inv_cnt, sbl_ref[...])
    cur = jnp.dot(cur, lW_ref[...], preferred_element_type=_F32) + lb_ref[...]
    logits = jnp.dot(cur, clsW_ref[...], preferred_element_type=_F32)
    logits = logits + clsb_ref[...]
    m = jnp.max(logits, axis=-1, keepdims=True)
    lse = jnp.log(jnp.sum(jnp.exp(logits - m), axis=-1, keepdims=True)) + m
    out_ref[...] = logits - lse


def _full(shape):
    return pl.BlockSpec(shape, lambda i: tuple(0 for _ in shape))


def _rows(shape):
    return pl.BlockSpec(shape, lambda i: (i,) + tuple(0 for _ in shape[1:]))


def _tc_call(body, n_grid, in_arrays, in_specs, out_shapes, out_specs):
    return pl.pallas_call(
        body,
        grid=(n_grid,),
        in_specs=in_specs,
        out_specs=out_specs,
        out_shape=out_shapes,
    )(*in_arrays)


def kernel(x, edge_index, edge_weight, pre_W0, pre_b0, cheb_W0_0, cheb_W1_0,
           cheb_b_0, sage_Wl_0, sage_bl_0, sage_Wr_0, lin_W_0, lin_b_0,
           pre_W1, pre_b1, cheb_W0_1, cheb_W1_1, cheb_b_1, sage_Wl_1,
           sage_bl_1, sage_Wr_1, lin_W_1, lin_b_1, cls_W, cls_b):
    n, f_in = x.shape
    e = edge_index.shape[1]
    cpt = e // (_NW * _CHUNK)
    assert e == _NW * cpt * _CHUNK and n % 2000 == 0 and n % (16 * 125) == 0

    src3 = edge_index[0].reshape(_NW, cpt, _CHUNK)
    dst3 = edge_index[1].reshape(_NW, cpt, _CHUNK)
    ew3 = edge_weight.reshape(_NW, cpt, _CHUNK)

    dc = _sc_degcnt(src3, dst3, ew3, n)
    dcT = jnp.transpose(dc, (2, 0, 1)).reshape(n, 4)

    r = 1000
    g = n // r
    b1 = lambda a: a.reshape(1, -1)

    y0, pq0 = _tc_call(
        _tcA_body, g,
        [x, dcT, pre_W0, b1(pre_b0), cheb_W0_0, cheb_W1_0, b1(cheb_b_0),
         sage_Wl_0, sage_Wr_0],
        [_rows((r, f_in)), _rows((r, 4)), _full((f_in, 64)), _full((1, 64)),
         _full((64, 64)), _full((64, 64)), _full((1, 64)), _full((64, 64)),
         _full((64, 64))],
        [jax.ShapeDtypeStruct((n, 128), _F32),
         jax.ShapeDtypeStruct((n, 128), _F32)],
        [_rows((r, 128)), _rows((r, 128))],
    )

    s0 = _sc_edge_pass(src3, dst3, ew3, y0)

    y1, pq1 = _tc_call(
        _tcB_body, g,
        [pq0, s0, dcT, b1(sage_bl_0), lin_W_0, b1(lin_b_0),
         pre_W1, b1(pre_b1), cheb_W0_1, cheb_W1_1, b1(cheb_b_1),
         sage_Wl_1, sage_Wr_1],
        [_rows((r, 128)), pl.BlockSpec((_NC, r, 128), lambda i: (0, i, 0)),
         _rows((r, 4)), _full((1, 64)), _full((64, 64)), _full((1, 64)),
         _full((64, 64)), _full((1, 64)), _full((64, 64)), _full((64, 64)),
         _full((1, 64)), _full((64, 64)), _full((64, 64))],
        [jax.ShapeDtypeStruct((n, 128), _F32),
         jax.ShapeDtypeStruct((n, 128), _F32)],
        [_rows((r, 128)), _rows((r, 128))],
    )

    s1 = _sc_edge_pass(src3, dst3, ew3, y1)

    out = _tc_call(
        _tcC_body, g,
        [pq1, s1, dcT, b1(sage_bl_1), lin_W_1, b1(lin_b_1), cls_W,
         b1(cls_b)],
        [_rows((r, 128)), pl.BlockSpec((_NC, r, 128), lambda i: (0, i, 0)),
         _rows((r, 4)), _full((1, 64)), _full((64, 64)), _full((1, 64)),
         _full((64, 32)), _full((1, 32))],
        jax.ShapeDtypeStruct((n, 32), _F32),
        _rows((r, 32)),
    )
    return out


# speedup vs baseline: 13.1666x; 13.1666x over previous
import jax
import jax.numpy as jnp
from jax import lax
from jax.experimental import pallas as pl
from jax.experimental.pallas import tpu as pltpu
from jax.experimental.pallas import tpu_sc as plsc

_NC = 2
_NS = 16
_NW = _NC * _NS
_CHUNK = 80
_F32 = jnp.float32


def _degcnt_body(src_hbm, dst_hbm, ew_hbm, out_hbm,
                 src_b, dst_b, ew_b, ones_v, zb_v, deg_s, cnt_s):
    n = out_hbm.shape[2]
    nb, bsz = src_hbm.shape[1], src_hbm.shape[2]
    c = lax.axis_index("c")
    s = lax.axis_index("s")
    w = s * _NC + c

    @pl.when(s == 0)
    def _zero():
        z16 = jnp.zeros((16,), _F32)

        def zfill(i, _):
            zb_v[pl.ds(i * 16, 16)] = z16
            return ()

        lax.fori_loop(0, 2000 // 16, zfill, ())
        for t in range(n // 2000):
            pltpu.sync_copy(zb_v, deg_s.at[pl.ds(2000 * t, 2000)])
            pltpu.sync_copy(zb_v, cnt_s.at[pl.ds(2000 * t, 2000)])

    one16 = jnp.ones((16,), _F32)
    for t in range(_CHUNK // 16):
        ones_v[pl.ds(16 * t, 16)] = one16
    plsc.subcore_barrier()

    def batch(jb, _):
        pltpu.sync_copy(src_hbm.at[w, jb], src_b)
        pltpu.sync_copy(dst_hbm.at[w, jb], dst_b)
        pltpu.sync_copy(ew_hbm.at[w, jb], ew_b)

        def chunk(jj, _):
            pltpu.sync_copy(ew_b.at[jj], deg_s.at[src_b.at[jj]], add=True)
            pltpu.sync_copy(ones_v, cnt_s.at[dst_b.at[jj]], add=True)
            return ()

        lax.fori_loop(0, bsz, chunk, ())
        return ()

    lax.fori_loop(0, nb, batch, ())
    plsc.subcore_barrier()

    @pl.when(s == 0)
    def _writeback():
        pltpu.sync_copy(deg_s, out_hbm.at[c, 0])
        pltpu.sync_copy(cnt_s, out_hbm.at[c, 1])


def _edge_body(src_hbm, dst_hbm, ew_hbm, y_hbm, out_hbm,
               src_b, dst_b, ew_b, rows_v, acc_s, sem):
    n = y_hbm.shape[0]
    nb = src_hbm.shape[1]
    rpt = n // _NS
    c = lax.axis_index("c")
    s = lax.axis_index("s")
    w = s * _NC + c

    z16 = jnp.zeros((16,), _F32)

    def zfill(i, _):
        for q in range(8):
            rows_v[i, pl.ds(q * 16, 16)] = z16
        return ()

    lax.fori_loop(0, _CHUNK, zfill, ())
    nz = rpt // _CHUNK
    for t in range(nz):
        pltpu.sync_copy(rows_v, acc_s.at[pl.ds(rpt * s + _CHUNK * t, _CHUNK)])
    rem = rpt - nz * _CHUNK
    if rem:
        pltpu.sync_copy(rows_v.at[pl.ds(0, rem)],
                        acc_s.at[pl.ds(rpt * s + nz * _CHUNK, rem)])
    plsc.subcore_barrier()

    def batch(jb, _):
        pltpu.sync_copy(src_hbm.at[w, jb], src_b)
        pltpu.sync_copy(dst_hbm.at[w, jb], dst_b)
        pltpu.sync_copy(ew_hbm.at[w, jb], ew_b)

        def chunk(jj, _):
            pltpu.async_copy(y_hbm.at[src_b.at[jj]], rows_v, sem).wait()
            for g in range(_CHUNK // 16):
                ew16 = ew_b[jj, pl.ds(g * 16, 16)]
                for kk in range(16):
                    bc = ew16.at[jnp.full((16,), kk, jnp.int32)].get(
                        mode="promise_in_bounds")
                    i = g * 16 + kk
                    for q in range(4):
                        sl = pl.ds(q * 16, 16)
                        rows_v[i, sl] = rows_v[i, sl] * bc
            pltpu.sync_copy(rows_v, acc_s.at[dst_b.at[jj]], add=True)
            return ()

        lax.fori_loop(0, src_hbm.shape[2], chunk, ())
        return ()

    lax.fori_loop(0, nb, batch, ())
    plsc.subcore_barrier()
    wrows = 632
    last = n - wrows * (_NS - 1)

    @pl.when(s < _NS - 1)
    def _wb():
        pltpu.sync_copy(acc_s.at[pl.ds(wrows * s, wrows)],
                        out_hbm.at[c, pl.ds(wrows * s, wrows)])

    @pl.when(s == _NS - 1)
    def _wb_last():
        pltpu.sync_copy(acc_s.at[pl.ds(wrows * s, last)],
                        out_hbm.at[c, pl.ds(wrows * s, last)])


def _sc_degcnt(src3, dst3, ew3, n):
    bsz, chunk = src3.shape[2], src3.shape[3]
    mesh = plsc.VectorSubcoreMesh(core_axis_name="c", subcore_axis_name="s")
    fn = pl.kernel(
        _degcnt_body,
        out_type=jax.ShapeDtypeStruct((_NC, 2, n), _F32),
        mesh=mesh,
        scratch_types=[
            pltpu.VMEM((bsz, chunk), jnp.int32),
            pltpu.VMEM((bsz, chunk), jnp.int32),
            pltpu.VMEM((bsz, chunk), _F32),
            pltpu.VMEM((chunk,), _F32),
            pltpu.VMEM((2000,), _F32),
            pltpu.VMEM_SHARED((n,), _F32),
            pltpu.VMEM_SHARED((n,), _F32),
        ],
    )
    return fn(src3, dst3, ew3)


def _sc_edge_pass(src3, dst3, ew3, y):
    n = y.shape[0]
    bsz, chunk = src3.shape[2], src3.shape[3]
    mesh = plsc.VectorSubcoreMesh(core_axis_name="c", subcore_axis_name="s")
    fn = pl.kernel(
        _edge_body,
        out_type=jax.ShapeDtypeStruct((_NC, n, 128), _F32),
        mesh=mesh,
        scratch_types=[
            pltpu.VMEM((bsz, chunk), jnp.int32),
            pltpu.VMEM((bsz, chunk), jnp.int32),
            pltpu.VMEM((bsz, chunk), _F32),
            pltpu.VMEM((chunk, 128), _F32),
            pltpu.VMEM_SHARED((n, 128), _F32),
            pltpu.SemaphoreType.DMA,
        ],
    )
    return fn(src3, dst3, ew3, y)


def _leaky(v):
    return jnp.where(v >= 0, v, 0.01 * v)


def _dis_invcnt(dc):
    deg = dc[:, 0:1] + dc[:, 2:3]
    cnt = dc[:, 1:2] + dc[:, 3:4]
    dis = jnp.where(deg > 0, lax.rsqrt(jnp.where(deg > 0, deg, 1.0)), 0.0)
    inv_cnt = 1.0 / jnp.clip(cnt, 1.0, None)
    return dis, inv_cnt


def _tcpre_body(x_ref, dc_ref, pW_ref, pb_ref, cW0_ref, cW1_ref, cb_ref,
                sWl_ref, sWr_ref, y_ref, pq_ref):
    dis, _ = _dis_invcnt(dc_ref[...])
    xx = jnp.dot(x_ref[...], pW_ref[...], preferred_element_type=_F32)
    xx = xx + pb_ref[...]
    u = jnp.dot(xx, cW1_ref[...], preferred_element_type=_F32) * dis
    v = jnp.dot(xx, sWl_ref[...], preferred_element_type=_F32)
    y_ref[...] = jnp.concatenate([u, v], axis=1)
    p = jnp.dot(xx, cW0_ref[...], preferred_element_type=_F32) + cb_ref[...]
    q = jnp.dot(xx, sWr_ref[...], preferred_element_type=_F32)
    pq_ref[...] = jnp.concatenate([p, q], axis=1)


def _tccomb_body(pq_ref, s_ref, dc_ref, sbl_ref, lW_ref, lb_ref, cur_ref):
    dis, inv_cnt = _dis_invcnt(dc_ref[...])
    pq = pq_ref[...]
    S = s_ref[...]
    sa = S[0, :, :64] + S[1, :, :64]
    sb = S[0, :, 64:] + S[1, :, 64:]
    o1 = _leaky(pq[:, :64] - dis * sa)
    o2 = _leaky(inv_cnt * sb + sbl_ref[...] + pq[:, 64:])
    cur = jnp.dot(o1 + o2, lW_ref[...], preferred_element_type=_F32)
    cur_ref[...] = cur + lb_ref[...]


def _tccls_body(cur_ref, clsW_ref, clsb_ref, out_ref):
    logits = jnp.dot(cur_ref[...], clsW_ref[...],
                     preferred_element_type=_F32)
    logits = logits + clsb_ref[...]
    m = jnp.max(logits, axis=-1, keepdims=True)
    lse = jnp.log(jnp.sum(jnp.exp(logits - m), axis=-1, keepdims=True)) + m
    out_ref[...] = logits - lse


def _full(shape):
    return pl.BlockSpec(shape, lambda i: tuple(0 for _ in shape))


def _rows(shape):
    return pl.BlockSpec(shape, lambda i: (i,) + tuple(0 for _ in shape[1:]))


def _tc_call(body, n_grid, in_arrays, in_specs, out_shapes, out_specs):
    return pl.pallas_call(
        body,
        grid=(n_grid,),
        in_specs=in_specs,
        out_specs=out_specs,
        out_shape=out_shapes,
    )(*in_arrays)


def kernel(x, edge_index, edge_weight, pre_W0, pre_b0, cheb_W0_0, cheb_W1_0,
           cheb_b_0, sage_Wl_0, sage_bl_0, sage_Wr_0, lin_W_0, lin_b_0,
           pre_W1, pre_b1, cheb_W0_1, cheb_W1_1, cheb_b_1, sage_Wl_1,
           sage_bl_1, sage_Wr_1, lin_W_1, lin_b_1, cls_W, cls_b):
    n, f_in = x.shape
    e = edge_index.shape[1]
    cpt3 = e // (_NW * _CHUNK)
    assert e == _NW * cpt3 * _CHUNK and n % 2000 == 0
    bs3 = 5
    assert cpt3 % bs3 == 0

    sh3 = (_NW, cpt3 // bs3, bs3, _CHUNK)
    src3 = edge_index[0].reshape(sh3)
    dst3 = edge_index[1].reshape(sh3)
    ew3 = edge_weight.reshape(sh3)

    dc = _sc_degcnt(src3, dst3, ew3, n)
    dcT = jnp.transpose(dc, (2, 0, 1)).reshape(n, 4)

    r = 1000
    g = n // r
    b1 = lambda a: a.reshape(1, -1)

    cur = x
    f = f_in
    layers = [
        (pre_W0, pre_b0, cheb_W0_0, cheb_W1_0, cheb_b_0, sage_Wl_0,
         sage_bl_0, sage_Wr_0, lin_W_0, lin_b_0),
        (pre_W1, pre_b1, cheb_W0_1, cheb_W1_1, cheb_b_1, sage_Wl_1,
         sage_bl_1, sage_Wr_1, lin_W_1, lin_b_1),
    ]
    for pW, pb, cW0, cW1, cb, sWl, sbl, sWr, lW, lb in layers:
        y, pq = _tc_call(
            _tcpre_body, g,
            [cur, dcT, pW, b1(pb), cW0, cW1, b1(cb), sWl, sWr],
            [_rows((r, f)), _rows((r, 4)), _full((f, 64)), _full((1, 64)),
             _full((64, 64)), _full((64, 64)), _full((1, 64)),
             _full((64, 64)), _full((64, 64))],
            [jax.ShapeDtypeStruct((n, 128), _F32),
             jax.ShapeDtypeStruct((n, 128), _F32)],
            [_rows((r, 128)), _rows((r, 128))],
        )
        s = _sc_edge_pass(src3, dst3, ew3, y)
        cur = _tc_call(
            _tccomb_body, g,
            [pq, s, dcT, b1(sbl), lW, b1(lb)],
            [_rows((r, 128)), pl.BlockSpec((_NC, r, 128), lambda i: (0, i, 0)),
             _rows((r, 4)), _full((1, 64)), _full((64, 64)), _full((1, 64))],
            jax.ShapeDtypeStruct((n, 64), _F32),
            _rows((r, 64)),
        )
        f = 64

    out = _tc_call(
        _tccls_body, g,
        [cur, cls_W, b1(cls_b)],
        [_rows((r, 64)), _full((64, 32)), _full((1, 32))],
        jax.ShapeDtypeStruct((n, 32), _F32),
        _rows((r, 32)),
    )
    return out


# speedup vs baseline: 18.2472x; 1.3859x over previous
import jax
import jax.numpy as jnp
from jax import lax
from jax.experimental import pallas as pl
from jax.experimental.pallas import tpu as pltpu
from jax.experimental.pallas import tpu_sc as plsc

_NC = 2
_NS = 16
_NW = _NC * _NS
_CHUNK = 80
_F32 = jnp.float32


def _degcnt_body(src_hbm, dst_hbm, ew_hbm, out_hbm,
                 src_b, dst_b, ew_b, ones_v, zb_v, deg_s, cnt_s):
    n = out_hbm.shape[2]
    nb, bsz = src_hbm.shape[1], src_hbm.shape[2]
    c = lax.axis_index("c")
    s = lax.axis_index("s")
    w = s * _NC + c

    @pl.when(s == 0)
    def _zero():
        z16 = jnp.zeros((16,), _F32)

        def zfill(i, _):
            zb_v[pl.ds(i * 16, 16)] = z16
            return ()

        lax.fori_loop(0, 2000 // 16, zfill, ())
        for t in range(n // 2000):
            pltpu.sync_copy(zb_v, deg_s.at[pl.ds(2000 * t, 2000)])
            pltpu.sync_copy(zb_v, cnt_s.at[pl.ds(2000 * t, 2000)])

    one16 = jnp.ones((16,), _F32)
    for t in range(_CHUNK // 16):
        ones_v[pl.ds(16 * t, 16)] = one16
    plsc.subcore_barrier()

    def batch(jb, _):
        pltpu.sync_copy(src_hbm.at[w, jb], src_b)
        pltpu.sync_copy(dst_hbm.at[w, jb], dst_b)
        pltpu.sync_copy(ew_hbm.at[w, jb], ew_b)

        def chunk(jj, _):
            pltpu.sync_copy(ew_b.at[jj], deg_s.at[src_b.at[jj]], add=True)
            pltpu.sync_copy(ones_v, cnt_s.at[dst_b.at[jj]], add=True)
            return ()

        lax.fori_loop(0, bsz, chunk, ())
        return ()

    lax.fori_loop(0, nb, batch, ())
    plsc.subcore_barrier()

    @pl.when(s == 0)
    def _writeback():
        pltpu.sync_copy(deg_s, out_hbm.at[c, 0])
        pltpu.sync_copy(cnt_s, out_hbm.at[c, 1])


def _edge_body(src_hbm, dst_hbm, ew_hbm, y_hbm, out_hbm,
               src_b, dst_b, ew_b, rows_a, rows_b,
               acc_s, semga, semgb, semsa, semsb):
    n = y_hbm.shape[0]
    nb = src_hbm.shape[1]
    rpt = n // _NS
    c = lax.axis_index("c")
    s = lax.axis_index("s")
    w = s * _NC + c

    z16 = jnp.zeros((16,), _F32)

    def zfill(i, _):
        for q in range(8):
            rows_a[i, pl.ds(q * 16, 16)] = z16
        return ()

    lax.fori_loop(0, _CHUNK, zfill, ())
    nz = rpt // _CHUNK
    for t in range(nz):
        pltpu.sync_copy(rows_a, acc_s.at[pl.ds(rpt * s + _CHUNK * t, _CHUNK)])
    rem = rpt - nz * _CHUNK
    if rem:
        pltpu.sync_copy(rows_a.at[pl.ds(0, rem)],
                        acc_s.at[pl.ds(rpt * s + nz * _CHUNK, rem)])
    plsc.subcore_barrier()

    bsz = dst_hbm.shape[2]

    def scale(rows, jj):
        for g in range(_CHUNK // 16):
            ew16 = ew_b[pl.ds(jj * _CHUNK + g * 16, 16)]
            for kk in range(16):
                bc = ew16.at[jnp.full((16,), kk, jnp.int32)].get(
                    mode="promise_in_bounds")
                i = g * 16 + kk
                for q in range(4):
                    sl = pl.ds(q * 16, 16)
                    rows[i, sl] = rows[i, sl] * bc

    def startg(rows, jj, sg):
        pltpu.async_copy(y_hbm.at[src_b.at[pl.ds(jj * _CHUNK, _CHUNK)]],
                         rows, sg)

    def waitg(rows, sg):
        pltpu.make_async_copy(y_hbm.at[src_b.at[pl.ds(0, _CHUNK)]],
                              rows, sg).wait()

    def starts(rows, jj, ss):
        pltpu.async_copy(rows, acc_s.at[dst_b.at[jj]], ss, add=True)

    def waits(rows, ss):
        pltpu.make_async_copy(rows, acc_s.at[dst_b.at[0]], ss).wait()

    def batch(jb, _):
        pltpu.sync_copy(src_hbm.at[w, jb], src_b)
        pltpu.sync_copy(dst_hbm.at[w, jb], dst_b)
        pltpu.sync_copy(ew_hbm.at[w, jb], ew_b)
        startg(rows_a, 0, semga)
        startg(rows_b, 1, semgb)

        def pair(jp, _):
            a = 2 * jp
            b = a + 1
            waitg(rows_a, semga)
            scale(rows_a, a)
            starts(rows_a, a, semsa)
            waitg(rows_b, semgb)
            scale(rows_b, b)
            starts(rows_b, b, semsb)

            @pl.when(a + 2 < bsz)
            def _ga():
                waits(rows_a, semsa)
                startg(rows_a, a + 2, semga)

            @pl.when(b + 2 < bsz)
            def _gb():
                waits(rows_b, semsb)
                startg(rows_b, b + 2, semgb)

            return ()

        lax.fori_loop(0, bsz // 2, pair, ())
        if bsz % 2:
            waitg(rows_a, semga)
            scale(rows_a, bsz - 1)
            starts(rows_a, bsz - 1, semsa)
        waits(rows_a, semsa)
        waits(rows_b, semsb)
        return ()

    lax.fori_loop(0, nb, batch, ())
    plsc.subcore_barrier()
    wrows = 632
    last = n - wrows * (_NS - 1)

    @pl.when(s < _NS - 1)
    def _wb():
        pltpu.sync_copy(acc_s.at[pl.ds(wrows * s, wrows)],
                        out_hbm.at[c, pl.ds(wrows * s, wrows)])

    @pl.when(s == _NS - 1)
    def _wb_last():
        pltpu.sync_copy(acc_s.at[pl.ds(wrows * s, last)],
                        out_hbm.at[c, pl.ds(wrows * s, last)])


def _sc_degcnt(src3, dst3, ew3, n):
    bsz, chunk = src3.shape[2], src3.shape[3]
    mesh = plsc.VectorSubcoreMesh(core_axis_name="c", subcore_axis_name="s")
    fn = pl.kernel(
        _degcnt_body,
        out_type=jax.ShapeDtypeStruct((_NC, 2, n), _F32),
        mesh=mesh,
        scratch_types=[
            pltpu.VMEM((bsz, chunk), jnp.int32),
            pltpu.VMEM((bsz, chunk), jnp.int32),
            pltpu.VMEM((bsz, chunk), _F32),
            pltpu.VMEM((chunk,), _F32),
            pltpu.VMEM((2000,), _F32),
            pltpu.VMEM_SHARED((n,), _F32),
            pltpu.VMEM_SHARED((n,), _F32),
        ],
    )
    return fn(src3, dst3, ew3)


def _sc_edge_pass(srcf, dst4, ewf, y):
    n = y.shape[0]
    bsz, chunk = dst4.shape[2], dst4.shape[3]
    mesh = plsc.VectorSubcoreMesh(core_axis_name="c", subcore_axis_name="s")
    fn = pl.kernel(
        _edge_body,
        out_type=jax.ShapeDtypeStruct((_NC, n, 128), _F32),
        mesh=mesh,
        scratch_types=[
            pltpu.VMEM((bsz * chunk,), jnp.int32),
            pltpu.VMEM((bsz, chunk), jnp.int32),
            pltpu.VMEM((bsz * chunk,), _F32),
            pltpu.VMEM((chunk, 128), _F32),
            pltpu.VMEM((chunk, 128), _F32),
            pltpu.VMEM_SHARED((n, 128), _F32),
            pltpu.SemaphoreType.DMA,
            pltpu.SemaphoreType.DMA,
            pltpu.SemaphoreType.DMA,
            pltpu.SemaphoreType.DMA,
        ],
    )
    return fn(srcf, dst4, ewf, y)


def _leaky(v):
    return jnp.where(v >= 0, v, 0.01 * v)


def _dis_invcnt(dc):
    deg = dc[:, 0:1] + dc[:, 2:3]
    cnt = dc[:, 1:2] + dc[:, 3:4]
    dis = jnp.where(deg > 0, lax.rsqrt(jnp.where(deg > 0, deg, 1.0)), 0.0)
    inv_cnt = 1.0 / jnp.clip(cnt, 1.0, None)
    return dis, inv_cnt


def _tcpre_body(x_ref, dc_ref, pW_ref, pb_ref, cW0_ref, cW1_ref, cb_ref,
                sWl_ref, sWr_ref, y_ref, pq_ref):
    dis, _ = _dis_invcnt(dc_ref[...])
    xx = jnp.dot(x_ref[...], pW_ref[...], preferred_element_type=_F32)
    xx = xx + pb_ref[...]
    u = jnp.dot(xx, cW1_ref[...], preferred_element_type=_F32) * dis
    v = jnp.dot(xx, sWl_ref[...], preferred_element_type=_F32)
    y_ref[...] = jnp.concatenate([u, v], axis=1)
    p = jnp.dot(xx, cW0_ref[...], preferred_element_type=_F32) + cb_ref[...]
    q = jnp.dot(xx, sWr_ref[...], preferred_element_type=_F32)
    pq_ref[...] = jnp.concatenate([p, q], axis=1)


def _tccomb_body(pq_ref, s_ref, dc_ref, sbl_ref, lW_ref, lb_ref, cur_ref):
    dis, inv_cnt = _dis_invcnt(dc_ref[...])
    pq = pq_ref[...]
    S = s_ref[...]
    sa = S[0, :, :64] + S[1, :, :64]
    sb = S[0, :, 64:] + S[1, :, 64:]
    o1 = _leaky(pq[:, :64] - dis * sa)
    o2 = _leaky(inv_cnt * sb + sbl_ref[...] + pq[:, 64:])
    cur = jnp.dot(o1 + o2, lW_ref[...], preferred_element_type=_F32)
    cur_ref[...] = cur + lb_ref[...]


def _tccls_body(cur_ref, clsW_ref, clsb_ref, out_ref):
    logits = jnp.dot(cur_ref[...], clsW_ref[...],
                     preferred_element_type=_F32)
    logits = logits + clsb_ref[...]
    m = jnp.max(logits, axis=-1, keepdims=True)
    lse = jnp.log(jnp.sum(jnp.exp(logits - m), axis=-1, keepdims=True)) + m
    out_ref[...] = logits - lse


def _full(shape):
    return pl.BlockSpec(shape, lambda i: tuple(0 for _ in shape))


def _rows(shape):
    return pl.BlockSpec(shape, lambda i: (i,) + tuple(0 for _ in shape[1:]))


def _tc_call(body, n_grid, in_arrays, in_specs, out_shapes, out_specs):
    return pl.pallas_call(
        body,
        grid=(n_grid,),
        in_specs=in_specs,
        out_specs=out_specs,
        out_shape=out_shapes,
    )(*in_arrays)


def kernel(x, edge_index, edge_weight, pre_W0, pre_b0, cheb_W0_0, cheb_W1_0,
           cheb_b_0, sage_Wl_0, sage_bl_0, sage_Wr_0, lin_W_0, lin_b_0,
           pre_W1, pre_b1, cheb_W0_1, cheb_W1_1, cheb_b_1, sage_Wl_1,
           sage_bl_1, sage_Wr_1, lin_W_1, lin_b_1, cls_W, cls_b):
    n, f_in = x.shape
    e = edge_index.shape[1]
    cpt3 = e // (_NW * _CHUNK)
    assert e == _NW * cpt3 * _CHUNK and n % 2000 == 0
    bs3 = 5
    assert cpt3 % bs3 == 0

    sh3 = (_NW, cpt3 // bs3, bs3, _CHUNK)
    src3 = edge_index[0].reshape(sh3)
    dst3 = edge_index[1].reshape(sh3)
    ew3 = edge_weight.reshape(sh3)

    bse = 25
    assert cpt3 % bse == 0
    nbe = cpt3 // bse
    src_e = edge_index[0].reshape(_NW, nbe, bse * _CHUNK)
    dst_e = edge_index[1].reshape(_NW, nbe, bse, _CHUNK)
    ew_e = edge_weight.reshape(_NW, nbe, bse * _CHUNK)

    dc = _sc_degcnt(src3, dst3, ew3, n)
    dcT = jnp.transpose(dc, (2, 0, 1)).reshape(n, 4)

    r = 1000
    g = n // r
    b1 = lambda a: a.reshape(1, -1)

    cur = x
    f = f_in
    layers = [
        (pre_W0, pre_b0, cheb_W0_0, cheb_W1_0, cheb_b_0, sage_Wl_0,
         sage_bl_0, sage_Wr_0, lin_W_0, lin_b_0),
        (pre_W1, pre_b1, cheb_W0_1, cheb_W1_1, cheb_b_1, sage_Wl_1,
         sage_bl_1, sage_Wr_1, lin_W_1, lin_b_1),
    ]
    for pW, pb, cW0, cW1, cb, sWl, sbl, sWr, lW, lb in layers:
        y, pq = _tc_call(
            _tcpre_body, g,
            [cur, dcT, pW, b1(pb), cW0, cW1, b1(cb), sWl, sWr],
            [_rows((r, f)), _rows((r, 4)), _full((f, 64)), _full((1, 64)),
             _full((64, 64)), _full((64, 64)), _full((1, 64)),
             _full((64, 64)), _full((64, 64))],
            [jax.ShapeDtypeStruct((n, 128), _F32),
             jax.ShapeDtypeStruct((n, 128), _F32)],
            [_rows((r, 128)), _rows((r, 128))],
        )
        s = _sc_edge_pass(src_e, dst_e, ew_e, y)
        cur = _tc_call(
            _tccomb_body, g,
            [pq, s, dcT, b1(sbl), lW, b1(lb)],
            [_rows((r, 128)), pl.BlockSpec((_NC, r, 128), lambda i: (0, i, 0)),
             _rows((r, 4)), _full((1, 64)), _full((64, 64)), _full((1, 64))],
            jax.ShapeDtypeStruct((n, 64), _F32),
            _rows((r, 64)),
        )
        f = 64

    out = _tc_call(
        _tccls_body, g,
        [cur, cls_W, b1(cls_b)],
        [_rows((r, 64)), _full((64, 32)), _full((1, 32))],
        jax.ShapeDtypeStruct((n, 32), _F32),
        _rows((r, 32)),
    )
    return out


# speedup vs baseline: 20.2206x; 1.1082x over previous
import jax
import jax.numpy as jnp
from jax import lax
from jax.experimental import pallas as pl
from jax.experimental.pallas import tpu as pltpu
from jax.experimental.pallas import tpu_sc as plsc

_NC = 2
_NS = 16
_NW = _NC * _NS
_CHUNK = 80
_F32 = jnp.float32


def _degcnt_body(src_hbm, dst_hbm, ew_hbm, out_hbm,
                 src_b, dst_b, ew_b, ones_v, zb_v, deg_s, cnt_s):
    n = out_hbm.shape[2]
    nb, bsz = src_hbm.shape[1], src_hbm.shape[2]
    c = lax.axis_index("c")
    s = lax.axis_index("s")
    w = s * _NC + c

    @pl.when(s == 0)
    def _zero():
        z16 = jnp.zeros((16,), _F32)

        def zfill(i, _):
            zb_v[pl.ds(i * 16, 16)] = z16
            return ()

        lax.fori_loop(0, 2000 // 16, zfill, ())
        for t in range(n // 2000):
            pltpu.sync_copy(zb_v, deg_s.at[pl.ds(2000 * t, 2000)])
            pltpu.sync_copy(zb_v, cnt_s.at[pl.ds(2000 * t, 2000)])

    one16 = jnp.ones((16,), _F32)
    for t in range(_CHUNK // 16):
        ones_v[pl.ds(16 * t, 16)] = one16
    plsc.subcore_barrier()

    def batch(jb, _):
        pltpu.sync_copy(src_hbm.at[w, jb], src_b)
        pltpu.sync_copy(dst_hbm.at[w, jb], dst_b)
        pltpu.sync_copy(ew_hbm.at[w, jb], ew_b)

        def chunk(jj, _):
            pltpu.sync_copy(ew_b.at[jj], deg_s.at[src_b.at[jj]], add=True)
            pltpu.sync_copy(ones_v, cnt_s.at[dst_b.at[jj]], add=True)
            return ()

        lax.fori_loop(0, bsz, chunk, ())
        return ()

    lax.fori_loop(0, nb, batch, ())
    plsc.subcore_barrier()

    @pl.when(s == 0)
    def _writeback():
        pltpu.sync_copy(deg_s, out_hbm.at[c, 0])
        pltpu.sync_copy(cnt_s, out_hbm.at[c, 1])


def _edge_body(src_hbm, dst_hbm, ew_hbm, y_hbm, out_hbm,
               src_b, dst_b, ew_b, rows_a, rows_b,
               acc_s, semga, semgb, semsa, semsb):
    n = y_hbm.shape[0]
    nb = src_hbm.shape[1]
    rpt = n // _NS
    c = lax.axis_index("c")
    s = lax.axis_index("s")
    w = s * _NC + c

    z16 = jnp.zeros((16,), _F32)

    def zfill(i, _):
        for q in range(8):
            rows_a[i, pl.ds(q * 16, 16)] = z16
        return ()

    lax.fori_loop(0, _CHUNK, zfill, ())
    nz = rpt // _CHUNK
    for t in range(nz):
        pltpu.sync_copy(rows_a, acc_s.at[pl.ds(rpt * s + _CHUNK * t, _CHUNK)])
    rem = rpt - nz * _CHUNK
    if rem:
        pltpu.sync_copy(rows_a.at[pl.ds(0, rem)],
                        acc_s.at[pl.ds(rpt * s + nz * _CHUNK, rem)])
    plsc.subcore_barrier()

    bsz = dst_hbm.shape[2]

    def scale(rows, jj):
        for g in range(_CHUNK // 16):
            ew16 = ew_b[pl.ds(jj * _CHUNK + g * 16, 16)]
            for kk in range(16):
                bc = ew16.at[jnp.full((16,), kk, jnp.int32)].get(
                    mode="promise_in_bounds")
                i = g * 16 + kk
                for q in range(4):
                    sl = pl.ds(q * 16, 16)
                    rows[i, sl] = rows[i, sl] * bc

    def startg(rows, jj, sg):
        pltpu.async_copy(y_hbm.at[src_b.at[pl.ds(jj * _CHUNK, _CHUNK)]],
                         rows, sg)

    def waitg(rows, sg):
        pltpu.make_async_copy(y_hbm.at[src_b.at[pl.ds(0, _CHUNK)]],
                              rows, sg).wait()

    def starts(rows, jj, ss):
        pltpu.async_copy(rows, acc_s.at[dst_b.at[jj]], ss, add=True)

    def waits(rows, ss):
        pltpu.make_async_copy(rows, acc_s.at[dst_b.at[0]], ss).wait()

    def batch(jb, _):
        pltpu.sync_copy(src_hbm.at[w, jb], src_b)
        pltpu.sync_copy(dst_hbm.at[w, jb], dst_b)
        pltpu.sync_copy(ew_hbm.at[w, jb], ew_b)
        startg(rows_a, 0, semga)
        startg(rows_b, 1, semgb)

        def pair(jp, _):
            a = 2 * jp
            b = a + 1
            waitg(rows_a, semga)
            scale(rows_a, a)
            starts(rows_a, a, semsa)
            waitg(rows_b, semgb)
            scale(rows_b, b)
            starts(rows_b, b, semsb)

            @pl.when(a + 2 < bsz)
            def _ga():
                waits(rows_a, semsa)
                startg(rows_a, a + 2, semga)

            @pl.when(b + 2 < bsz)
            def _gb():
                waits(rows_b, semsb)
                startg(rows_b, b + 2, semgb)

            return ()

        lax.fori_loop(0, bsz // 2, pair, ())
        if bsz % 2:
            waitg(rows_a, semga)
            scale(rows_a, bsz - 1)
            starts(rows_a, bsz - 1, semsa)
        waits(rows_a, semsa)
        waits(rows_b, semsb)
        return ()

    lax.fori_loop(0, nb, batch, ())
    plsc.subcore_barrier()
    wrows = 632
    last = n - wrows * (_NS - 1)

    @pl.when(s < _NS - 1)
    def _wb():
        pltpu.sync_copy(acc_s.at[pl.ds(wrows * s, wrows)],
                        out_hbm.at[c, pl.ds(wrows * s, wrows)])

    @pl.when(s == _NS - 1)
    def _wb_last():
        pltpu.sync_copy(acc_s.at[pl.ds(wrows * s, last)],
                        out_hbm.at[c, pl.ds(wrows * s, last)])


def _sc_degcnt(src3, dst3, ew3, n):
    bsz, chunk = src3.shape[2], src3.shape[3]
    mesh = plsc.VectorSubcoreMesh(core_axis_name="c", subcore_axis_name="s")
    fn = pl.kernel(
        _degcnt_body,
        out_type=jax.ShapeDtypeStruct((_NC, 2, n), _F32),
        mesh=mesh,
        scratch_types=[
            pltpu.VMEM((bsz, chunk), jnp.int32),
            pltpu.VMEM((bsz, chunk), jnp.int32),
            pltpu.VMEM((bsz, chunk), _F32),
            pltpu.VMEM((chunk,), _F32),
            pltpu.VMEM((2000,), _F32),
            pltpu.VMEM_SHARED((n,), _F32),
            pltpu.VMEM_SHARED((n,), _F32),
        ],
    )
    return fn(src3, dst3, ew3)


def _sc_edge_pass(srcf, dst4, ewf, y):
    n = y.shape[0]
    bsz, chunk = dst4.shape[2], dst4.shape[3]
    mesh = plsc.VectorSubcoreMesh(core_axis_name="c", subcore_axis_name="s")
    fn = pl.kernel(
        _edge_body,
        out_type=jax.ShapeDtypeStruct((_NC, n, 128), _F32),
        mesh=mesh,
        scratch_types=[
            pltpu.VMEM((bsz * chunk,), jnp.int32),
            pltpu.VMEM((bsz, chunk), jnp.int32),
            pltpu.VMEM((bsz * chunk,), _F32),
            pltpu.VMEM((chunk, 128), _F32),
            pltpu.VMEM((chunk, 128), _F32),
            pltpu.VMEM_SHARED((n, 128), _F32),
            pltpu.SemaphoreType.DMA,
            pltpu.SemaphoreType.DMA,
            pltpu.SemaphoreType.DMA,
            pltpu.SemaphoreType.DMA,
        ],
    )
    return fn(srcf, dst4, ewf, y)


def _leaky(v):
    return jnp.where(v >= 0, v, 0.01 * v)


def _dis_invcnt(dc):
    deg = dc[:, 0:1] + dc[:, 2:3]
    cnt = dc[:, 1:2] + dc[:, 3:4]
    dis = jnp.where(deg > 0, lax.rsqrt(jnp.where(deg > 0, deg, 1.0)), 0.0)
    inv_cnt = 1.0 / jnp.clip(cnt, 1.0, None)
    return dis, inv_cnt


def _tcpre_body(x_ref, dc_ref, pW_ref, pb_ref, cW0_ref, cW1_ref, cb_ref,
                sWl_ref, sWr_ref, y_ref, pq_ref):
    dis, _ = _dis_invcnt(dc_ref[...])
    xx = jnp.dot(x_ref[...], pW_ref[...], preferred_element_type=_F32)
    xx = xx + pb_ref[...]
    u = jnp.dot(xx, cW1_ref[...], preferred_element_type=_F32) * dis
    v = jnp.dot(xx, sWl_ref[...], preferred_element_type=_F32)
    y_ref[...] = jnp.concatenate([u, v], axis=1)
    p = jnp.dot(xx, cW0_ref[...], preferred_element_type=_F32) + cb_ref[...]
    q = jnp.dot(xx, sWr_ref[...], preferred_element_type=_F32)
    pq_ref[...] = jnp.concatenate([p, q], axis=1)


def _comb(pq_ref, s_ref, dis, inv_cnt, sbl_ref, lW_ref, lb_ref):
    pq = pq_ref[...]
    S = s_ref[...]
    sa = S[0, :, :64] + S[1, :, :64]
    sb = S[0, :, 64:] + S[1, :, 64:]
    o1 = _leaky(pq[:, :64] - dis * sa)
    o2 = _leaky(inv_cnt * sb + sbl_ref[...] + pq[:, 64:])
    cur = jnp.dot(o1 + o2, lW_ref[...], preferred_element_type=_F32)
    return cur + lb_ref[...]


def _tccombpre_body(pq_ref, s_ref, dc_ref, sbl_ref, lW_ref, lb_ref,
                    pW_ref, pb_ref, cW0_ref, cW1_ref, cb_ref,
                    sWl_ref, sWr_ref, y_ref, pq1_ref):
    dis, inv_cnt = _dis_invcnt(dc_ref[...])
    cur = _comb(pq_ref, s_ref, dis, inv_cnt, sbl_ref, lW_ref, lb_ref)
    xx = jnp.dot(cur, pW_ref[...], preferred_element_type=_F32) + pb_ref[...]
    u = jnp.dot(xx, cW1_ref[...], preferred_element_type=_F32) * dis
    v = jnp.dot(xx, sWl_ref[...], preferred_element_type=_F32)
    y_ref[...] = jnp.concatenate([u, v], axis=1)
    p = jnp.dot(xx, cW0_ref[...], preferred_element_type=_F32) + cb_ref[...]
    q = jnp.dot(xx, sWr_ref[...], preferred_element_type=_F32)
    pq1_ref[...] = jnp.concatenate([p, q], axis=1)


def _tccombcls_body(pq_ref, s_ref, dc_ref, sbl_ref, lW_ref, lb_ref,
                    clsW_ref, clsb_ref, out_ref):
    dis, inv_cnt = _dis_invcnt(dc_ref[...])
    cur = _comb(pq_ref, s_ref, dis, inv_cnt, sbl_ref, lW_ref, lb_ref)
    logits = jnp.dot(cur, clsW_ref[...], preferred_element_type=_F32)
    logits = logits + clsb_ref[...]
    m = jnp.max(logits, axis=-1, keepdims=True)
    lse = jnp.log(jnp.sum(jnp.exp(logits - m), axis=-1, keepdims=True)) + m
    out_ref[...] = logits - lse


def _full(shape):
    return pl.BlockSpec(shape, lambda i: tuple(0 for _ in shape))


def _rows(shape):
    return pl.BlockSpec(shape, lambda i: (i,) + tuple(0 for _ in shape[1:]))


def _tc_call(body, n_grid, in_arrays, in_specs, out_shapes, out_specs):
    return pl.pallas_call(
        body,
        grid=(n_grid,),
        in_specs=in_specs,
        out_specs=out_specs,
        out_shape=out_shapes,
    )(*in_arrays)


def kernel(x, edge_index, edge_weight, pre_W0, pre_b0, cheb_W0_0, cheb_W1_0,
           cheb_b_0, sage_Wl_0, sage_bl_0, sage_Wr_0, lin_W_0, lin_b_0,
           pre_W1, pre_b1, cheb_W0_1, cheb_W1_1, cheb_b_1, sage_Wl_1,
           sage_bl_1, sage_Wr_1, lin_W_1, lin_b_1, cls_W, cls_b):
    n, f_in = x.shape
    e = edge_index.shape[1]
    cpt3 = e // (_NW * _CHUNK)
    assert e == _NW * cpt3 * _CHUNK and n % 2000 == 0
    bs3 = 25
    assert cpt3 % bs3 == 0

    sh3 = (_NW, cpt3 // bs3, bs3, _CHUNK)
    src3 = edge_index[0].reshape(sh3)
    dst3 = edge_index[1].reshape(sh3)
    ew3 = edge_weight.reshape(sh3)

    bse = 25
    assert cpt3 % bse == 0
    nbe = cpt3 // bse
    src_e = edge_index[0].reshape(_NW, nbe, bse * _CHUNK)
    dst_e = edge_index[1].reshape(_NW, nbe, bse, _CHUNK)
    ew_e = edge_weight.reshape(_NW, nbe, bse * _CHUNK)

    dc = _sc_degcnt(src3, dst3, ew3, n)
    dcT = jnp.transpose(dc, (2, 0, 1)).reshape(n, 4)

    r = 1000
    g = n // r
    b1 = lambda a: a.reshape(1, -1)

    f64 = _full((64, 64))
    f1 = _full((1, 64))
    sspec = pl.BlockSpec((_NC, r, 128), lambda i: (0, i, 0))
    yshape = [jax.ShapeDtypeStruct((n, 128), _F32),
              jax.ShapeDtypeStruct((n, 128), _F32)]
    yspecs = [_rows((r, 128)), _rows((r, 128))]

    y0, pq0 = _tc_call(
        _tcpre_body, g,
        [x, dcT, pre_W0, b1(pre_b0), cheb_W0_0, cheb_W1_0, b1(cheb_b_0),
         sage_Wl_0, sage_Wr_0],
        [_rows((r, f_in)), _rows((r, 4)), _full((f_in, 64)), f1,
         f64, f64, f1, f64, f64],
        yshape, yspecs,
    )
    s0 = _sc_edge_pass(src_e, dst_e, ew_e, y0)
    y1, pq1 = _tc_call(
        _tccombpre_body, g,
        [pq0, s0, dcT, b1(sage_bl_0), lin_W_0, b1(lin_b_0),
         pre_W1, b1(pre_b1), cheb_W0_1, cheb_W1_1, b1(cheb_b_1),
         sage_Wl_1, sage_Wr_1],
        [_rows((r, 128)), sspec, _rows((r, 4)), f1, f64, f1,
         f64, f1, f64, f64, f1, f64, f64],
        yshape, yspecs,
    )
    s1 = _sc_edge_pass(src_e, dst_e, ew_e, y1)
    out = _tc_call(
        _tccombcls_body, g,
        [pq1, s1, dcT, b1(sage_bl_1), lin_W_1, b1(lin_b_1), cls_W,
         b1(cls_b)],
        [_rows((r, 128)), sspec, _rows((r, 4)), f1, f64, f1,
         _full((64, 32)), _full((1, 32))],
        jax.ShapeDtypeStruct((n, 32), _F32),
        _rows((r, 32)),
    )
    return out
